# baseline (speedup 1.0000x reference)
import functools

import jax
import jax.numpy as jnp
from jax.experimental import pallas as pl
from jax.experimental.pallas import tpu as pltpu
from jax.experimental.pallas import tpu_sc as plsc

_LEAF_START = 131071
_RANK = 32
_N8 = 262144
_TBLK = 4096
_NW = 32
_KROWS = 4
_CHUNK = _KROWS * 128


def _transform_body(t_ref, w1_ref, b1_ref, w2_ref, b2_ref, o_ref):
    i = pl.program_id(0)
    x = t_ref[...]
    xb = x.astype(jnp.bfloat16)
    h = jax.lax.dot_general(w1_ref[...], xb, (((1,), (0,)), ((), ())),
                            preferred_element_type=jnp.float32)
    h = jnp.maximum(h + b1_ref[...], 0.0).astype(jnp.bfloat16)
    h2 = jax.lax.dot_general(w2_ref[...], h, (((1,), (0,)), ((), ())),
                             preferred_element_type=jnp.float32)
    h2 = h2 + b2_ref[...]
    cols = i * _TBLK + jax.lax.broadcasted_iota(jnp.int32, (1, _TBLK), 1)
    out_t = jnp.where(cols < _LEAF_START, h2, x)
    out = out_t.T
    o_ref[...] = jnp.concatenate(
        [out, jnp.zeros((_TBLK, 2 * _RANK), jnp.float32)], axis=1)


def _transform_table(table_t, w1t, b1col, w2t, b2col):
    return pl.pallas_call(
        _transform_body,
        grid=(_N8 // _TBLK,),
        in_specs=[
            pl.BlockSpec((2 * _RANK, _TBLK), lambda i: (0, i)),
            pl.BlockSpec((2 * _RANK, 2 * _RANK), lambda i: (0, 0)),
            pl.BlockSpec((2 * _RANK, 1), lambda i: (0, 0)),
            pl.BlockSpec((2 * _RANK, 2 * _RANK), lambda i: (0, 0)),
            pl.BlockSpec((2 * _RANK, 1), lambda i: (0, 0)),
        ],
        out_specs=pl.BlockSpec((_TBLK, 128), lambda i: (i, 0)),
        out_shape=jax.ShapeDtypeStruct((_N8, 128), jnp.float32),
    )(table_t, w1t, b1col, w2t, b2col)


def _sc_gather(tp, idx3d):
    m = idx3d.shape[0] * _CHUNK
    n_chunks = m // _CHUNK
    per_w = n_chunks // _NW
    mesh = plsc.VectorSubcoreMesh(core_axis_name="c", subcore_axis_name="s")

    @functools.partial(
        pl.kernel,
        mesh=mesh,
        out_type=jax.ShapeDtypeStruct((m, 128), jnp.float32),
        scratch_types=[
            pltpu.VMEM((per_w, _KROWS, 128), jnp.int32),
            pltpu.VMEM((_CHUNK // 2, 128), jnp.float32),
            pltpu.VMEM((_CHUNK // 2, 128), jnp.float32),
            pltpu.SemaphoreType.DMA,
            pltpu.SemaphoreType.DMA,
            pltpu.SemaphoreType.DMA,
            pltpu.SemaphoreType.DMA,
        ],
        compiler_params=pltpu.CompilerParams(use_tc_tiling_on_sc=True),
    )
    def k(tp_hbm, idx_hbm, g_hbm, idx_v, ra_v, rb_v, sa, sb, swa, swb):
        wid = jax.lax.axis_index("s") * 2 + jax.lax.axis_index("c")
        base = wid * per_w
        half = _CHUNK // 2
        pltpu.sync_copy(idx_hbm.at[pl.ds(base, per_w)], idx_v)

        def fire_a(ci):
            for j in range(2):
                pltpu.async_copy(tp_hbm.at[idx_v.at[ci, j]],
                                 ra_v.at[pl.ds(j * 128, 128)], sa)

        def wait_a(ci):
            for j in range(2):
                pltpu.make_async_copy(tp_hbm.at[idx_v.at[ci, j]],
                                      ra_v.at[pl.ds(j * 128, 128)], sa).wait()

        fire_a(0)

        @pl.loop(0, per_w)
        def _(ci):
            c = base + ci
            for j in range(2, 4):
                pltpu.async_copy(tp_hbm.at[idx_v.at[ci, j]],
                                 rb_v.at[pl.ds((j - 2) * 128, 128)], sb)
            wait_a(ci)
            wba = pltpu.async_copy(
                ra_v, g_hbm.at[pl.ds(c * _CHUNK, half)], swa)
            for j in range(2, 4):
                pltpu.make_async_copy(tp_hbm.at[idx_v.at[ci, j]],
                                      rb_v.at[pl.ds((j - 2) * 128, 128)],
                                      sb).wait()
            wbb = pltpu.async_copy(
                rb_v, g_hbm.at[pl.ds(c * _CHUNK + half, half)], swb)
            wba.wait()
            cin = jnp.minimum(ci + 1, per_w - 1)
            fire_a(cin)
            wbb.wait()

        wait_a(per_w - 1)

    return k(tp, idx3d)


_BBLK = 32
_NSLICE = 4


def _extract_body(g_ref, o0_ref, o1_ref):
    x = g_ref[...]
    for q in range(_BBLK):
        xt = x[q * 512:(q + 1) * 512, :2 * _RANK].T
        o0_ref[q] = xt[:_RANK]
        o1_ref[q] = xt[_RANK:]


def _extract_first_body(g_ref, o0_ref, o1_ref):
    _extract_body(g_ref, o0_ref, o1_ref)


def _extract_slice_body(g_ref, p0_ref, p1_ref, o0_ref, o1_ref):
    del p0_ref, p1_ref
    _extract_body(g_ref, o0_ref, o1_ref)


def _extract_first(g0, b, n, bs):
    return pl.pallas_call(
        _extract_first_body,
        grid=(bs // _BBLK,),
        compiler_params=pltpu.CompilerParams(
            dimension_semantics=("arbitrary",)),
        in_specs=[pl.BlockSpec((_BBLK * n, 128), lambda i: (i, 0))],
        out_specs=[
            pl.BlockSpec((_BBLK, _RANK, n), lambda i: (i, 0, 0)),
            pl.BlockSpec((_BBLK, _RANK, n), lambda i: (i, 0, 0)),
        ],
        out_shape=[
            jax.ShapeDtypeStruct((b, _RANK, n), jnp.float32),
            jax.ShapeDtypeStruct((b, _RANK, n), jnp.float32),
        ],
    )(g0)


def _extract_slice(g_s, p0, p1, off_blocks, n, bs):
    off = off_blocks
    return pl.pallas_call(
        _extract_slice_body,
        grid=(bs // _BBLK,),
        in_specs=[
            pl.BlockSpec((_BBLK * n, 128), lambda i: (i, 0)),
            pl.BlockSpec((1, 8, 128), lambda i: (0, 0, 0)),
            pl.BlockSpec((1, 8, 128), lambda i: (0, 0, 0)),
        ],
        out_specs=[
            pl.BlockSpec((_BBLK, _RANK, n), lambda i, o=off: (o + i, 0, 0)),
            pl.BlockSpec((_BBLK, _RANK, n), lambda i, o=off: (o + i, 0, 0)),
        ],
        out_shape=[
            jax.ShapeDtypeStruct(p0.shape, jnp.float32),
            jax.ShapeDtypeStruct(p1.shape, jnp.float32),
        ],
        input_output_aliases={1: 0, 2: 1},
    )(g_s, p0, p1)


def kernel(nodeIdx, table, W1_0, b1_0, W2_0, b2_0, W1_1, b1_1, W2_1, b2_1):
    r = _RANK
    w1bd = jnp.zeros((2 * r, 2 * r), jnp.float32)
    w1bd = w1bd.at[:r, :r].set(W1_0).at[r:, r:].set(W1_1)
    w2bd = jnp.zeros((2 * r, 2 * r), jnp.float32)
    w2bd = w2bd.at[:r, :r].set(W2_0).at[r:, r:].set(W2_1)
    w1t = w1bd.T.astype(jnp.bfloat16)
    w2t = w2bd.T.astype(jnp.bfloat16)
    b1col = jnp.concatenate([b1_0, b1_1]).reshape(2 * r, 1)
    b2col = jnp.concatenate([b2_0, b2_1]).reshape(2 * r, 1)

    tp = _transform_table(table.T, w1t, b1col, w2t, b2col)

    b, n = nodeIdx.shape
    idx3d = nodeIdx.reshape(b * n // _CHUNK, _KROWS, 128)
    slice_b = (320, 320, 320, 64)
    cpb = n // _CHUNK
    offs = [sum(slice_b[:s]) for s in range(len(slice_b))]
    gs = [_sc_gather(tp, idx3d[o * cpb:(o + sb) * cpb])
          for o, sb in zip(offs, slice_b)]
    o0t, o1t = _extract_first(gs[0], b, n, slice_b[0])
    for s in range(1, len(slice_b)):
        o0t, o1t = _extract_slice(gs[s], o0t, o1t, offs[s] // _BBLK, n,
                                  slice_b[s])
    return (jnp.transpose(o0t, (0, 2, 1)), jnp.transpose(o1t, (0, 2, 1)))


# speedup vs baseline: 1.0095x; 1.0095x over previous
import functools

import jax
import jax.numpy as jnp
from jax.experimental import pallas as pl
from jax.experimental.pallas import tpu as pltpu
from jax.experimental.pallas import tpu_sc as plsc

_LEAF_START = 131071
_RANK = 32
_N8 = 262144
_TBLK = 4096
_NW = 32
_KROWS = 4
_CHUNK = _KROWS * 128


def _transform_body(t_ref, w1_ref, b1_ref, w2_ref, b2_ref, o_ref):
    i = pl.program_id(0)
    x = t_ref[...]
    xb = x.astype(jnp.bfloat16)
    h = jax.lax.dot_general(w1_ref[...], xb, (((1,), (0,)), ((), ())),
                            preferred_element_type=jnp.float32)
    h = jnp.maximum(h + b1_ref[...], 0.0).astype(jnp.bfloat16)
    h2 = jax.lax.dot_general(w2_ref[...], h, (((1,), (0,)), ((), ())),
                             preferred_element_type=jnp.float32)
    h2 = h2 + b2_ref[...]
    cols = i * _TBLK + jax.lax.broadcasted_iota(jnp.int32, (1, _TBLK), 1)
    out_t = jnp.where(cols < _LEAF_START, h2, x)
    out = out_t.T
    o_ref[...] = jnp.concatenate(
        [out, jnp.zeros((_TBLK, 2 * _RANK), jnp.float32)], axis=1)


def _transform_table(table_t, w1t, b1col, w2t, b2col):
    return pl.pallas_call(
        _transform_body,
        grid=(_N8 // _TBLK,),
        in_specs=[
            pl.BlockSpec((2 * _RANK, _TBLK), lambda i: (0, i)),
            pl.BlockSpec((2 * _RANK, 2 * _RANK), lambda i: (0, 0)),
            pl.BlockSpec((2 * _RANK, 1), lambda i: (0, 0)),
            pl.BlockSpec((2 * _RANK, 2 * _RANK), lambda i: (0, 0)),
            pl.BlockSpec((2 * _RANK, 1), lambda i: (0, 0)),
        ],
        out_specs=pl.BlockSpec((_TBLK, 128), lambda i: (i, 0)),
        out_shape=jax.ShapeDtypeStruct((_N8, 128), jnp.float32),
    )(table_t, w1t, b1col, w2t, b2col)


def _sc_gather(tp, idx3d):
    m = idx3d.shape[0] * _CHUNK
    n_chunks = m // _CHUNK
    per_w = n_chunks // _NW
    mesh = plsc.VectorSubcoreMesh(core_axis_name="c", subcore_axis_name="s")

    @functools.partial(
        pl.kernel,
        mesh=mesh,
        out_type=jax.ShapeDtypeStruct((m, 128), jnp.float32),
        scratch_types=[
            pltpu.VMEM((per_w, _KROWS, 128), jnp.int32),
            pltpu.VMEM((_CHUNK, 128), jnp.float32),
            pltpu.SemaphoreType.DMA,
        ],
        compiler_params=pltpu.CompilerParams(use_tc_tiling_on_sc=True),
    )
    def k(tp_hbm, idx_hbm, g_hbm, idx_v, r_v, sem):
        wid = jax.lax.axis_index("s") * 2 + jax.lax.axis_index("c")
        base = wid * per_w
        pltpu.sync_copy(idx_hbm.at[pl.ds(base, per_w)], idx_v)

        @pl.loop(0, per_w)
        def _(ci):
            c = base + ci
            copies = []
            for j in range(_KROWS):
                copies.append(pltpu.async_copy(
                    tp_hbm.at[idx_v.at[ci, j]],
                    r_v.at[pl.ds(j * 128, 128)], sem))
            for cp in copies:
                cp.wait()
            pltpu.sync_copy(r_v, g_hbm.at[pl.ds(c * _CHUNK, _CHUNK)])

    return k(tp, idx3d)


_BBLK = 32
_NSLICE = 4


def _extract_body(g_ref, o0_ref, o1_ref):
    x = g_ref[...]
    for q in range(_BBLK):
        xt = x[q * 512:(q + 1) * 512, :2 * _RANK].T
        o0_ref[q] = xt[:_RANK]
        o1_ref[q] = xt[_RANK:]


def _extract_first_body(g_ref, o0_ref, o1_ref):
    _extract_body(g_ref, o0_ref, o1_ref)


def _extract_slice_body(g_ref, p0_ref, p1_ref, o0_ref, o1_ref):
    del p0_ref, p1_ref
    _extract_body(g_ref, o0_ref, o1_ref)


def _extract_first(g0, b, n, bs):
    return pl.pallas_call(
        _extract_first_body,
        grid=(bs // _BBLK,),
        compiler_params=pltpu.CompilerParams(
            dimension_semantics=("arbitrary",)),
        in_specs=[pl.BlockSpec((_BBLK * n, 128), lambda i: (i, 0))],
        out_specs=[
            pl.BlockSpec((_BBLK, _RANK, n), lambda i: (i, 0, 0)),
            pl.BlockSpec((_BBLK, _RANK, n), lambda i: (i, 0, 0)),
        ],
        out_shape=[
            jax.ShapeDtypeStruct((b, _RANK, n), jnp.float32),
            jax.ShapeDtypeStruct((b, _RANK, n), jnp.float32),
        ],
    )(g0)


def _extract_slice(g_s, p0, p1, off_blocks, n, bs):
    off = off_blocks
    return pl.pallas_call(
        _extract_slice_body,
        grid=(bs // _BBLK,),
        in_specs=[
            pl.BlockSpec((_BBLK * n, 128), lambda i: (i, 0)),
            pl.BlockSpec((1, 8, 128), lambda i: (0, 0, 0)),
            pl.BlockSpec((1, 8, 128), lambda i: (0, 0, 0)),
        ],
        out_specs=[
            pl.BlockSpec((_BBLK, _RANK, n), lambda i, o=off: (o + i, 0, 0)),
            pl.BlockSpec((_BBLK, _RANK, n), lambda i, o=off: (o + i, 0, 0)),
        ],
        out_shape=[
            jax.ShapeDtypeStruct(p0.shape, jnp.float32),
            jax.ShapeDtypeStruct(p1.shape, jnp.float32),
        ],
        input_output_aliases={1: 0, 2: 1},
    )(g_s, p0, p1)


def kernel(nodeIdx, table, W1_0, b1_0, W2_0, b2_0, W1_1, b1_1, W2_1, b2_1):
    r = _RANK
    w1bd = jnp.zeros((2 * r, 2 * r), jnp.float32)
    w1bd = w1bd.at[:r, :r].set(W1_0).at[r:, r:].set(W1_1)
    w2bd = jnp.zeros((2 * r, 2 * r), jnp.float32)
    w2bd = w2bd.at[:r, :r].set(W2_0).at[r:, r:].set(W2_1)
    w1t = w1bd.T.astype(jnp.bfloat16)
    w2t = w2bd.T.astype(jnp.bfloat16)
    b1col = jnp.concatenate([b1_0, b1_1]).reshape(2 * r, 1)
    b2col = jnp.concatenate([b2_0, b2_1]).reshape(2 * r, 1)

    tp = _transform_table(table.T, w1t, b1col, w2t, b2col)

    b, n = nodeIdx.shape
    idx3d = nodeIdx.reshape(b * n // _CHUNK, _KROWS, 128)
    slice_b = (128,) * 8
    cpb = n // _CHUNK
    offs = [sum(slice_b[:s]) for s in range(len(slice_b))]
    gs = [_sc_gather(tp, idx3d[o * cpb:(o + sb) * cpb])
          for o, sb in zip(offs, slice_b)]
    o0t, o1t = _extract_first(gs[0], b, n, slice_b[0])
    for s in range(1, len(slice_b)):
        o0t, o1t = _extract_slice(gs[s], o0t, o1t, offs[s] // _BBLK, n,
                                  slice_b[s])
    return (jnp.transpose(o0t, (0, 2, 1)), jnp.transpose(o1t, (0, 2, 1)))


# speedup vs baseline: 1.0578x; 1.0478x over previous
import functools

import jax
import jax.numpy as jnp
from jax.experimental import pallas as pl
from jax.experimental.pallas import tpu as pltpu
from jax.experimental.pallas import tpu_sc as plsc

_LEAF_START = 131071
_RANK = 32
_N8 = 262144
_TBLK = 8192
_NW = 32
_KROWS = 4
_CHUNK = _KROWS * 128


def _transform_body(t_ref, w1_ref, b1_ref, w2_ref, b2_ref, o_ref):
    i = pl.program_id(0)
    x = t_ref[...]
    xb = x.astype(jnp.bfloat16)
    h = jax.lax.dot_general(w1_ref[...], xb, (((1,), (0,)), ((), ())),
                            preferred_element_type=jnp.float32)
    h = jnp.maximum(h + b1_ref[...], 0.0).astype(jnp.bfloat16)
    h2 = jax.lax.dot_general(w2_ref[...], h, (((1,), (0,)), ((), ())),
                             preferred_element_type=jnp.float32)
    h2 = h2 + b2_ref[...]
    cols = i * _TBLK + jax.lax.broadcasted_iota(jnp.int32, (1, _TBLK), 1)
    out_t = jnp.where(cols < _LEAF_START, h2, x)
    out = out_t.T
    o_ref[...] = jnp.concatenate(
        [out, jnp.zeros((_TBLK, 2 * _RANK), jnp.float32)], axis=1)


def _transform_table(table_t, w1t, b1col, w2t, b2col):
    return pl.pallas_call(
        _transform_body,
        grid=(_N8 // _TBLK,),
        in_specs=[
            pl.BlockSpec((2 * _RANK, _TBLK), lambda i: (0, i)),
            pl.BlockSpec((2 * _RANK, 2 * _RANK), lambda i: (0, 0)),
            pl.BlockSpec((2 * _RANK, 1), lambda i: (0, 0)),
            pl.BlockSpec((2 * _RANK, 2 * _RANK), lambda i: (0, 0)),
            pl.BlockSpec((2 * _RANK, 1), lambda i: (0, 0)),
        ],
        out_specs=pl.BlockSpec((_TBLK, 128), lambda i: (i, 0)),
        out_shape=jax.ShapeDtypeStruct((_N8, 128), jnp.float32),
    )(table_t, w1t, b1col, w2t, b2col)


def _sc_gather(tp, idx3d):
    m = idx3d.shape[0] * _CHUNK
    n_chunks = m // _CHUNK
    per_w = n_chunks // _NW
    mesh = plsc.VectorSubcoreMesh(core_axis_name="c", subcore_axis_name="s")

    @functools.partial(
        pl.kernel,
        mesh=mesh,
        out_type=jax.ShapeDtypeStruct((m, 128), jnp.float32),
        scratch_types=[
            pltpu.VMEM((per_w, _KROWS, 128), jnp.int32),
            pltpu.VMEM((_CHUNK, 128), jnp.float32),
            pltpu.SemaphoreType.DMA,
        ],
        compiler_params=pltpu.CompilerParams(use_tc_tiling_on_sc=True),
    )
    def k(tp_hbm, idx_hbm, g_hbm, idx_v, r_v, sem):
        wid = jax.lax.axis_index("s") * 2 + jax.lax.axis_index("c")
        base = wid * per_w
        pltpu.sync_copy(idx_hbm.at[pl.ds(base, per_w)], idx_v)

        @pl.loop(0, per_w)
        def _(ci):
            c = base + ci
            copies = []
            for j in range(_KROWS):
                copies.append(pltpu.async_copy(
                    tp_hbm.at[idx_v.at[ci, j]],
                    r_v.at[pl.ds(j * 128, 128)], sem))
            for cp in copies:
                cp.wait()
            pltpu.sync_copy(r_v, g_hbm.at[pl.ds(c * _CHUNK, _CHUNK)])

    return k(tp, idx3d)


_BBLK = 32
_NSLICE = 4


def _extract_body(g_ref, o0_ref, o1_ref):
    x = g_ref[...]
    for q in range(_BBLK):
        xt = x[q * 512:(q + 1) * 512, :2 * _RANK].T
        o0_ref[q] = xt[:_RANK]
        o1_ref[q] = xt[_RANK:]


def _extract_first_body(g_ref, o0_ref, o1_ref):
    _extract_body(g_ref, o0_ref, o1_ref)


def _extract_slice_body(g_ref, p0_ref, p1_ref, o0_ref, o1_ref):
    del p0_ref, p1_ref
    _extract_body(g_ref, o0_ref, o1_ref)


def _extract_first(g0, b, n, bs):
    return pl.pallas_call(
        _extract_first_body,
        grid=(bs // _BBLK,),
        compiler_params=pltpu.CompilerParams(
            dimension_semantics=("arbitrary",)),
        in_specs=[pl.BlockSpec((_BBLK * n, 128), lambda i: (i, 0))],
        out_specs=[
            pl.BlockSpec((_BBLK, _RANK, n), lambda i: (i, 0, 0)),
            pl.BlockSpec((_BBLK, _RANK, n), lambda i: (i, 0, 0)),
        ],
        out_shape=[
            jax.ShapeDtypeStruct((b, _RANK, n), jnp.float32),
            jax.ShapeDtypeStruct((b, _RANK, n), jnp.float32),
        ],
    )(g0)


def _extract_slice(g_s, p0, p1, off_blocks, n, bs):
    off = off_blocks
    return pl.pallas_call(
        _extract_slice_body,
        grid=(bs // _BBLK,),
        in_specs=[
            pl.BlockSpec((_BBLK * n, 128), lambda i: (i, 0)),
            pl.BlockSpec((1, 8, 128), lambda i: (0, 0, 0)),
            pl.BlockSpec((1, 8, 128), lambda i: (0, 0, 0)),
        ],
        out_specs=[
            pl.BlockSpec((_BBLK, _RANK, n), lambda i, o=off: (o + i, 0, 0)),
            pl.BlockSpec((_BBLK, _RANK, n), lambda i, o=off: (o + i, 0, 0)),
        ],
        out_shape=[
            jax.ShapeDtypeStruct(p0.shape, jnp.float32),
            jax.ShapeDtypeStruct(p1.shape, jnp.float32),
        ],
        input_output_aliases={1: 0, 2: 1},
    )(g_s, p0, p1)


def kernel(nodeIdx, table, W1_0, b1_0, W2_0, b2_0, W1_1, b1_1, W2_1, b2_1):
    r = _RANK
    w1bd = jnp.zeros((2 * r, 2 * r), jnp.float32)
    w1bd = w1bd.at[:r, :r].set(W1_0).at[r:, r:].set(W1_1)
    w2bd = jnp.zeros((2 * r, 2 * r), jnp.float32)
    w2bd = w2bd.at[:r, :r].set(W2_0).at[r:, r:].set(W2_1)
    w1t = w1bd.T.astype(jnp.bfloat16)
    w2t = w2bd.T.astype(jnp.bfloat16)
    b1col = jnp.concatenate([b1_0, b1_1]).reshape(2 * r, 1)
    b2col = jnp.concatenate([b2_0, b2_1]).reshape(2 * r, 1)

    tp = _transform_table(table.T, w1t, b1col, w2t, b2col)

    b, n = nodeIdx.shape
    idx3d = nodeIdx.reshape(b * n // _CHUNK, _KROWS, 128)
    slice_b = (320, 320, 320, 64)
    cpb = n // _CHUNK
    offs = [sum(slice_b[:s]) for s in range(len(slice_b))]
    gs = [_sc_gather(tp, idx3d[o * cpb:(o + sb) * cpb])
          for o, sb in zip(offs, slice_b)]
    o0t, o1t = _extract_first(gs[0], b, n, slice_b[0])
    for s in range(1, len(slice_b)):
        o0t, o1t = _extract_slice(gs[s], o0t, o1t, offs[s] // _BBLK, n,
                                  slice_b[s])
    return (jnp.transpose(o0t, (0, 2, 1)), jnp.transpose(o1t, (0, 2, 1)))


# speedup vs baseline: 1.0768x; 1.0179x over previous
import functools

import jax
import jax.numpy as jnp
from jax.experimental import pallas as pl
from jax.experimental.pallas import tpu as pltpu
from jax.experimental.pallas import tpu_sc as plsc

_LEAF_START = 131071
_RANK = 32
_N8 = 262144
_TBLK = 16384
_NW = 32
_KROWS = 4
_CHUNK = _KROWS * 128


def _transform_body(t_ref, w1_ref, b1_ref, w2_ref, b2_ref, o_ref):
    i = pl.program_id(0)
    x = t_ref[...]
    xb = x.astype(jnp.bfloat16)
    h = jax.lax.dot_general(w1_ref[...], xb, (((1,), (0,)), ((), ())),
                            preferred_element_type=jnp.float32)
    h = jnp.maximum(h + b1_ref[...], 0.0).astype(jnp.bfloat16)
    h2 = jax.lax.dot_general(w2_ref[...], h, (((1,), (0,)), ((), ())),
                             preferred_element_type=jnp.float32)
    h2 = h2 + b2_ref[...]
    cols = i * _TBLK + jax.lax.broadcasted_iota(jnp.int32, (1, _TBLK), 1)
    out_t = jnp.where(cols < _LEAF_START, h2, x)
    out = out_t.T
    o_ref[...] = jnp.concatenate(
        [out, jnp.zeros((_TBLK, 2 * _RANK), jnp.float32)], axis=1)


def _transform_table(table_t, w1t, b1col, w2t, b2col):
    return pl.pallas_call(
        _transform_body,
        grid=(_N8 // _TBLK,),
        in_specs=[
            pl.BlockSpec((2 * _RANK, _TBLK), lambda i: (0, i)),
            pl.BlockSpec((2 * _RANK, 2 * _RANK), lambda i: (0, 0)),
            pl.BlockSpec((2 * _RANK, 1), lambda i: (0, 0)),
            pl.BlockSpec((2 * _RANK, 2 * _RANK), lambda i: (0, 0)),
            pl.BlockSpec((2 * _RANK, 1), lambda i: (0, 0)),
        ],
        out_specs=pl.BlockSpec((_TBLK, 128), lambda i: (i, 0)),
        out_shape=jax.ShapeDtypeStruct((_N8, 128), jnp.float32),
    )(table_t, w1t, b1col, w2t, b2col)


def _sc_gather(tp, idx3d):
    m = idx3d.shape[0] * _CHUNK
    n_chunks = m // _CHUNK
    per_w = n_chunks // _NW
    mesh = plsc.VectorSubcoreMesh(core_axis_name="c", subcore_axis_name="s")

    @functools.partial(
        pl.kernel,
        mesh=mesh,
        out_type=jax.ShapeDtypeStruct((m, 128), jnp.float32),
        scratch_types=[
            pltpu.VMEM((per_w, _KROWS, 128), jnp.int32),
            pltpu.VMEM((_CHUNK, 128), jnp.float32),
            pltpu.SemaphoreType.DMA,
        ],
        compiler_params=pltpu.CompilerParams(use_tc_tiling_on_sc=True),
    )
    def k(tp_hbm, idx_hbm, g_hbm, idx_v, r_v, sem):
        wid = jax.lax.axis_index("s") * 2 + jax.lax.axis_index("c")
        base = wid * per_w
        pltpu.sync_copy(idx_hbm.at[pl.ds(base, per_w)], idx_v)

        @pl.loop(0, per_w)
        def _(ci):
            c = base + ci
            copies = []
            for j in range(_KROWS):
                copies.append(pltpu.async_copy(
                    tp_hbm.at[idx_v.at[ci, j]],
                    r_v.at[pl.ds(j * 128, 128)], sem))
            for cp in copies:
                cp.wait()
            pltpu.sync_copy(r_v, g_hbm.at[pl.ds(c * _CHUNK, _CHUNK)])

    return k(tp, idx3d)


_BBLK = 32
_NSLICE = 4


def _extract_body(g_ref, o0_ref, o1_ref):
    x = g_ref[...]
    for q in range(_BBLK):
        xt = x[q * 512:(q + 1) * 512, :2 * _RANK].T
        o0_ref[q] = xt[:_RANK]
        o1_ref[q] = xt[_RANK:]


def _extract_first_body(g_ref, o0_ref, o1_ref):
    _extract_body(g_ref, o0_ref, o1_ref)


def _extract_slice_body(g_ref, p0_ref, p1_ref, o0_ref, o1_ref):
    del p0_ref, p1_ref
    _extract_body(g_ref, o0_ref, o1_ref)


def _extract_first(g0, b, n, bs):
    return pl.pallas_call(
        _extract_first_body,
        grid=(bs // _BBLK,),
        compiler_params=pltpu.CompilerParams(
            dimension_semantics=("arbitrary",)),
        in_specs=[pl.BlockSpec((_BBLK * n, 128), lambda i: (i, 0))],
        out_specs=[
            pl.BlockSpec((_BBLK, _RANK, n), lambda i: (i, 0, 0)),
            pl.BlockSpec((_BBLK, _RANK, n), lambda i: (i, 0, 0)),
        ],
        out_shape=[
            jax.ShapeDtypeStruct((b, _RANK, n), jnp.float32),
            jax.ShapeDtypeStruct((b, _RANK, n), jnp.float32),
        ],
    )(g0)


def _extract_slice(g_s, p0, p1, off_blocks, n, bs):
    off = off_blocks
    return pl.pallas_call(
        _extract_slice_body,
        grid=(bs // _BBLK,),
        in_specs=[
            pl.BlockSpec((_BBLK * n, 128), lambda i: (i, 0)),
            pl.BlockSpec((1, 8, 128), lambda i: (0, 0, 0)),
            pl.BlockSpec((1, 8, 128), lambda i: (0, 0, 0)),
        ],
        out_specs=[
            pl.BlockSpec((_BBLK, _RANK, n), lambda i, o=off: (o + i, 0, 0)),
            pl.BlockSpec((_BBLK, _RANK, n), lambda i, o=off: (o + i, 0, 0)),
        ],
        out_shape=[
            jax.ShapeDtypeStruct(p0.shape, jnp.float32),
            jax.ShapeDtypeStruct(p1.shape, jnp.float32),
        ],
        input_output_aliases={1: 0, 2: 1},
    )(g_s, p0, p1)


def kernel(nodeIdx, table, W1_0, b1_0, W2_0, b2_0, W1_1, b1_1, W2_1, b2_1):
    r = _RANK
    w1bd = jnp.zeros((2 * r, 2 * r), jnp.float32)
    w1bd = w1bd.at[:r, :r].set(W1_0).at[r:, r:].set(W1_1)
    w2bd = jnp.zeros((2 * r, 2 * r), jnp.float32)
    w2bd = w2bd.at[:r, :r].set(W2_0).at[r:, r:].set(W2_1)
    w1t = w1bd.T.astype(jnp.bfloat16)
    w2t = w2bd.T.astype(jnp.bfloat16)
    b1col = jnp.concatenate([b1_0, b1_1]).reshape(2 * r, 1)
    b2col = jnp.concatenate([b2_0, b2_1]).reshape(2 * r, 1)

    tp = _transform_table(table.T, w1t, b1col, w2t, b2col)

    b, n = nodeIdx.shape
    idx3d = nodeIdx.reshape(b * n // _CHUNK, _KROWS, 128)
    slice_b = (320, 320, 320, 64)
    cpb = n // _CHUNK
    offs = [sum(slice_b[:s]) for s in range(len(slice_b))]
    gs = [_sc_gather(tp, idx3d[o * cpb:(o + sb) * cpb])
          for o, sb in zip(offs, slice_b)]
    o0t, o1t = _extract_first(gs[0], b, n, slice_b[0])
    for s in range(1, len(slice_b)):
        o0t, o1t = _extract_slice(gs[s], o0t, o1t, offs[s] // _BBLK, n,
                                  slice_b[s])
    return (jnp.transpose(o0t, (0, 2, 1)), jnp.transpose(o1t, (0, 2, 1)))
